# baseline (speedup 1.0000x reference)
import functools

import jax
import jax.numpy as jnp
from jax import lax
from jax.experimental import pallas as pl
from jax.experimental.pallas import tpu as pltpu
from jax.experimental.pallas import tpu_sc as plsc

_E = 16
_T = 16384
_D = 2048
_BT = 512
_NBUF = 4
_NSTEPS = _T // _BT

_NC = 2
_NS = 16
_NW = _NC * _NS
_TPW = _T // _NW


def _mm_body(x_hbm, wt_ref, b_ref, o_ref, buf, sems):
    i = pl.program_id(0)

    @pl.when(i == 0)
    def _prime():
        for j in range(_NBUF - 1):
            pltpu.make_async_copy(
                x_hbm.at[pl.ds(j * _BT, _BT), :], buf.at[j], sems.at[j]
            ).start()

    slot = lax.rem(i, _NBUF)
    pltpu.make_async_copy(
        x_hbm.at[pl.ds(i * _BT, _BT), :], buf.at[slot], sems.at[slot]
    ).wait()

    nxt = i + _NBUF - 1

    @pl.when(nxt < _NSTEPS)
    def _fetch():
        nslot = lax.rem(nxt, _NBUF)
        pltpu.make_async_copy(
            x_hbm.at[pl.ds(nxt * _BT, _BT), :], buf.at[nslot], sems.at[nslot]
        ).start()

    o_ref[...] = (
        jnp.dot(buf[slot], wt_ref[...], preferred_element_type=jnp.float32)
        + b_ref[...]
    )


def _logits_tc(x, wt, b2):
    return pl.pallas_call(
        _mm_body,
        grid=(_NSTEPS,),
        in_specs=[
            pl.BlockSpec(memory_space=pl.ANY),
            pl.BlockSpec((_D, _E), lambda i: (0, 0)),
            pl.BlockSpec((1, _E), lambda i: (0, 0)),
        ],
        out_specs=pl.BlockSpec((_BT, _E), lambda i: (i, 0)),
        out_shape=jax.ShapeDtypeStruct((_T, _E), jnp.float32),
        scratch_shapes=[
            pltpu.VMEM((_NBUF, _BT, _D), jnp.float32),
            pltpu.SemaphoreType.DMA((_NBUF,)),
        ],
    )(x, wt, b2)


def _sc_gate(logits):
    mesh = plsc.VectorSubcoreMesh(core_axis_name="c", subcore_axis_name="s")

    @functools.partial(
        pl.kernel,
        mesh=mesh,
        out_type=jax.ShapeDtypeStruct((_T, _E), jnp.float32),
        scratch_types=[
            pltpu.VMEM((_TPW, _E), jnp.float32),
            pltpu.VMEM((_TPW, _E), jnp.float32),
        ],
        compiler_params=pltpu.CompilerParams(needs_layout_passes=False),
    )
    def k(logits_hbm, out_hbm, lv, ov):
        wid = lax.axis_index("s") * _NC + lax.axis_index("c")
        base = wid * _TPW
        pltpu.sync_copy(logits_hbm.at[pl.ds(base, _TPW)], lv)
        iota = lax.iota(jnp.int32, 16)
        efull = [jnp.full((16,), e, jnp.int32) for e in range(_E)]
        neginf = jnp.float32(-jnp.inf)

        ones16 = jnp.ones((16,), jnp.bool_)

        def body(g, carry):
            tok = g * 16 + iota

            def ld(e):
                return plsc.load_gather(lv, [tok, efull[e]])

            m1 = ld(0)
            for e in range(1, _E):
                m1 = jnp.maximum(m1, ld(e))
            nf = ones16
            m2 = None
            for e in range(_E):
                v = ld(e)
                eq = (v == m1) & nf
                nf = nf & (~eq)
                v2e = jnp.where(eq, neginf, v)
                m2 = v2e if m2 is None else jnp.maximum(m2, v2e)
            mx = jnp.maximum(m1, 0.0)
            nf1 = ones16
            nf2 = ones16
            s = None
            for e in range(_E):
                v = ld(e)
                eq1 = (v == m1) & nf1
                nf1 = nf1 & (~eq1)
                eq2 = (v == m2) & (~eq1) & nf2
                nf2 = nf2 & (~eq2)
                masked = jnp.where(eq1 | eq2, v, 0.0)
                ez = jnp.exp(masked - mx)
                plsc.store_scatter(ov, [tok, efull[e]], ez)
                s = ez if s is None else s + ez
            rinv = 1.0 / s
            for e in range(_E):
                z = plsc.load_gather(ov, [tok, efull[e]])
                plsc.store_scatter(ov, [tok, efull[e]], z * rinv)
            return carry

        lax.fori_loop(0, _TPW // 16, body, 0)
        pltpu.sync_copy(ov, out_hbm.at[pl.ds(base, _TPW)])

    return k(logits)


def kernel(x, W, b):
    wt = W.T
    b2 = b.reshape(1, _E)
    logits = _logits_tc(x, wt, b2)
    return _sc_gate(logits)


# speedup vs baseline: 1.3349x; 1.3349x over previous
import functools

import jax
import jax.numpy as jnp
from jax import lax
from jax.experimental import pallas as pl
from jax.experimental.pallas import tpu as pltpu
from jax.experimental.pallas import tpu_sc as plsc

_E = 16
_T = 16384
_D = 2048
_BT = 512
_NBUF = 4
_NSTEPS = _T // _BT

_NC = 2
_NS = 16
_NW = _NC * _NS
_TPW = _T // _NW


def _mm_body(x_hbm, wt_ref, b_ref, o_ref, buf, sems):
    i = pl.program_id(0)

    @pl.when(i == 0)
    def _prime():
        for j in range(_NBUF - 1):
            pltpu.make_async_copy(
                x_hbm.at[pl.ds(j * _BT, _BT), :], buf.at[j], sems.at[j]
            ).start()

    slot = lax.rem(i, _NBUF)
    pltpu.make_async_copy(
        x_hbm.at[pl.ds(i * _BT, _BT), :], buf.at[slot], sems.at[slot]
    ).wait()

    nxt = i + _NBUF - 1

    @pl.when(nxt < _NSTEPS)
    def _fetch():
        nslot = lax.rem(nxt, _NBUF)
        pltpu.make_async_copy(
            x_hbm.at[pl.ds(nxt * _BT, _BT), :], buf.at[nslot], sems.at[nslot]
        ).start()

    o_ref[...] = (
        jnp.dot(buf[slot], wt_ref[...], preferred_element_type=jnp.float32)
        + b_ref[...]
    )


def _logits_tc(x, wt, b2):
    return pl.pallas_call(
        _mm_body,
        grid=(_NSTEPS,),
        in_specs=[
            pl.BlockSpec(memory_space=pl.ANY),
            pl.BlockSpec((_D, _E), lambda i: (0, 0)),
            pl.BlockSpec((1, _E), lambda i: (0, 0)),
        ],
        out_specs=pl.BlockSpec((_BT, _E), lambda i: (i, 0)),
        out_shape=jax.ShapeDtypeStruct((_T, _E), jnp.float32),
        scratch_shapes=[
            pltpu.VMEM((_NBUF, _BT, _D), jnp.float32),
            pltpu.SemaphoreType.DMA((_NBUF,)),
        ],
    )(x, wt, b2)


def _sc_gate(logits):
    mesh = plsc.VectorSubcoreMesh(core_axis_name="c", subcore_axis_name="s")

    @functools.partial(
        pl.kernel,
        mesh=mesh,
        out_type=jax.ShapeDtypeStruct((_T, _E), jnp.float32),
        scratch_types=[
            pltpu.VMEM((_TPW, _E), jnp.float32),
            pltpu.VMEM((_TPW, _E), jnp.float32),
        ],
        compiler_params=pltpu.CompilerParams(needs_layout_passes=False),
    )
    def k(logits_hbm, out_hbm, lv, ov):
        wid = lax.axis_index("s") * _NC + lax.axis_index("c")
        base = wid * _TPW
        pltpu.sync_copy(logits_hbm.at[pl.ds(base, _TPW)], lv)
        iota = lax.iota(jnp.int32, 16)
        neginf = jnp.float32(-jnp.inf)

        def body(i, c):
            v = lv[i]
            m1 = jnp.max(v)
            i1 = plsc.all_reduce_ffs(v == m1)
            v2 = jnp.where(iota == i1, neginf, v)
            m2 = jnp.max(v2)
            i2 = plsc.all_reduce_ffs(v2 == m2)
            keep = (iota == i1) | (iota == i2)
            masked = jnp.where(keep, v, 0.0)
            e = jnp.exp(masked - jnp.maximum(m1, 0.0))
            ov[i] = e / jnp.sum(e)
            return c

        lax.fori_loop(0, _TPW, body, 0)
        pltpu.sync_copy(ov, out_hbm.at[pl.ds(base, _TPW)])

    return k(logits)


def kernel(x, W, b):
    wt = W.T
    b2 = b.reshape(1, _E)
    logits = _logits_tc(x, wt, b2)
    return _sc_gate(logits)


# speedup vs baseline: 1.3423x; 1.0055x over previous
import functools

import jax
import jax.numpy as jnp
from jax import lax
from jax.experimental import pallas as pl
from jax.experimental.pallas import tpu as pltpu
from jax.experimental.pallas import tpu_sc as plsc

_E = 16
_T = 16384
_D = 2048
_BT = 512
_NBUF = 4
_NSTEPS = _T // _BT

_NC = 2
_NS = 16
_NW = _NC * _NS
_TPW = _T // _NW


def _mm_body(x_hbm, wt_ref, b_ref, o_ref, buf, sems):
    i = pl.program_id(0)

    @pl.when(i == 0)
    def _prime():
        for j in range(_NBUF - 1):
            pltpu.make_async_copy(
                x_hbm.at[pl.ds(j * _BT, _BT), :], buf.at[j], sems.at[j]
            ).start()

    slot = lax.rem(i, _NBUF)
    pltpu.make_async_copy(
        x_hbm.at[pl.ds(i * _BT, _BT), :], buf.at[slot], sems.at[slot]
    ).wait()

    nxt = i + _NBUF - 1

    @pl.when(nxt < _NSTEPS)
    def _fetch():
        nslot = lax.rem(nxt, _NBUF)
        pltpu.make_async_copy(
            x_hbm.at[pl.ds(nxt * _BT, _BT), :], buf.at[nslot], sems.at[nslot]
        ).start()

    o_ref[...] = (
        jnp.dot(buf[slot], wt_ref[...], preferred_element_type=jnp.float32)
        + b_ref[...]
    )


def _logits_tc(x, wt, b2):
    return pl.pallas_call(
        _mm_body,
        grid=(_NSTEPS,),
        in_specs=[
            pl.BlockSpec(memory_space=pl.ANY),
            pl.BlockSpec((_D, _E), lambda i: (0, 0)),
            pl.BlockSpec((1, _E), lambda i: (0, 0)),
        ],
        out_specs=pl.BlockSpec((_BT, _E), lambda i: (i, 0)),
        out_shape=jax.ShapeDtypeStruct((_T, _E), jnp.float32),
        scratch_shapes=[
            pltpu.VMEM((_NBUF, _BT, _D), jnp.float32),
            pltpu.SemaphoreType.DMA((_NBUF,)),
        ],
    )(x, wt, b2)


def _sc_gate(logits):
    mesh = plsc.VectorSubcoreMesh(core_axis_name="c", subcore_axis_name="s")

    @functools.partial(
        pl.kernel,
        mesh=mesh,
        out_type=jax.ShapeDtypeStruct((_T, _E), jnp.float32),
        scratch_types=[
            pltpu.VMEM((_TPW, _E), jnp.float32),
            pltpu.VMEM((_TPW, _E), jnp.float32),
        ],
        compiler_params=pltpu.CompilerParams(needs_layout_passes=False),
    )
    def k(logits_hbm, out_hbm, lv, ov):
        wid = lax.axis_index("s") * _NC + lax.axis_index("c")
        base = wid * _TPW
        pltpu.sync_copy(logits_hbm.at[pl.ds(base, _TPW)], lv)
        iota = lax.iota(jnp.int32, 16)
        neginf = jnp.float32(-jnp.inf)

        @plsc.parallel_loop(0, _TPW, unroll=4)
        def body(i):
            v = lv[i]
            m1 = jnp.max(v)
            i1 = plsc.all_reduce_ffs(v == m1)
            v2 = jnp.where(iota == i1, neginf, v)
            m2 = jnp.max(v2)
            i2 = plsc.all_reduce_ffs(v2 == m2)
            keep = (iota == i1) | (iota == i2)
            masked = jnp.where(keep, v, 0.0)
            e = jnp.exp(masked - jnp.maximum(m1, 0.0))
            ov[i] = e / jnp.sum(e)
        pltpu.sync_copy(ov, out_hbm.at[pl.ds(base, _TPW)])

    return k(logits)


def kernel(x, W, b):
    wt = W.T
    b2 = b.reshape(1, _E)
    logits = _logits_tc(x, wt, b2)
    return _sc_gate(logits)
